# baseline (speedup 1.0000x reference)
import jax
import jax.numpy as jnp
from jax import lax
from jax.experimental import pallas as pl
from jax.experimental.pallas import tpu as pltpu
from jax.experimental.pallas import tpu_sc as plsc

_B, _S, _C = 4096, 50, 256
_NC, _NS = 2, 16
_NW = _NC * _NS
_RPW = _B // _NW
_T = 4
_STEPS = _RPW // _T
_IPS = _T * _S
_L = 16
_NVEC = (_IPS + _L - 1) // _L


def _sc_body(x_hbm, out_hbm, obuf, idxbuf, semo):
    c = lax.axis_index("c")
    s = lax.axis_index("s")
    wid = s * _NC + c
    base_row = wid * _RPW

    pltpu.sync_copy(x_hbm.at[pl.ds(wid * _RPW * _S, _RPW * _S)],
                    idxbuf.at[pl.ds(0, _RPW * _S)])

    lane = lax.iota(jnp.int32, _L)
    s50 = jnp.full((_L,), _S, jnp.int32)
    ones = jnp.full((_L,), 1.0, jnp.float32)
    zeros = jnp.full((_L,), 0.0, jnp.float32)
    tail = lane < jnp.full((_L,), _IPS - (_NVEC - 1) * _L, jnp.int32)

    for b in range(2):
        for a in range(_T):
            def _zrow(r, _, b=b, a=a):
                for k in range(_C // _L):
                    obuf[b, a, r, pl.ds(k * _L, _L)] = zeros
                return _
            lax.fori_loop(0, _S, _zrow, None)

    def _scatter(step, b, value):
        for v in range(_NVEC):
            k = lane + (v * _L)
            i0 = lax.div(k, s50)
            i1 = lax.rem(k, s50)
            i2 = idxbuf[pl.ds(step * _IPS + v * _L, _L)]
            mask = tail if v == _NVEC - 1 else None
            plsc.store_scatter(obuf.at[b], [i0, i1, i2], value, mask=mask)

    def _out_copy(j, b):
        return pltpu.make_async_copy(
            obuf.at[b], out_hbm.at[pl.ds(base_row + j * _T, _T)], semo.at[b]
        )

    def _do_step(j, b):
        @pl.when(j >= 2)
        def _reclaim():
            _out_copy(j - 2, b).wait()
            _scatter(j - 2, b, zeros)

        _scatter(j, b, ones)
        _out_copy(j, b).start()

    def _step(jj, _):
        _do_step(jj * 2, 0)
        _do_step(jj * 2 + 1, 1)
        return _

    lax.fori_loop(0, _STEPS // 2, _step, None)
    _out_copy(_STEPS - 2, 0).wait()
    _out_copy(_STEPS - 1, 1).wait()


def kernel(x):
    mesh = plsc.VectorSubcoreMesh(
        core_axis_name="c", subcore_axis_name="s",
        num_cores=_NC, num_subcores=_NS,
    )
    sc_onehot = pl.kernel(
        _sc_body,
        out_type=jax.ShapeDtypeStruct((_B, _S, _C), jnp.float32),
        mesh=mesh,
        scratch_types=[
            pltpu.VMEM((2, _T, _S, _C), jnp.float32),
            pltpu.VMEM((_RPW * _S + _L,), jnp.int32),
            pltpu.SemaphoreType.DMA((2,)),
        ],
        compiler_params=pltpu.CompilerParams(
            use_tc_tiling_on_sc=False, needs_layout_passes=False,
        ),
    )
    return sc_onehot(x.reshape(-1).astype(jnp.int32))


# speedup vs baseline: 1.9478x; 1.9478x over previous
import jax
import jax.numpy as jnp
from jax import lax
from jax.experimental import pallas as pl
from jax.experimental.pallas import tpu as pltpu
from jax.experimental.pallas import tpu_sc as plsc

_B, _S, _C = 4096, 50, 256
_NC, _NS = 2, 16
_NW = _NC * _NS
_RPW = _B // _NW
_T = 4
_STEPS = _RPW // _T
_IPS = _T * _S
_L = 16
_NVEC = (_IPS + _L - 1) // _L


def _sc_body(x_hbm, out_hbm, obuf, idxbuf, semo):
    c = lax.axis_index("c")
    s = lax.axis_index("s")
    wid = s * _NC + c
    base_row = wid * _RPW

    pltpu.sync_copy(x_hbm.at[pl.ds(wid * _RPW * _S, _RPW * _S)],
                    idxbuf.at[pl.ds(0, _RPW * _S)])

    lane = lax.iota(jnp.int32, _L)
    s50 = jnp.full((_L,), _S, jnp.int32)
    ones = jnp.full((_L,), 1.0, jnp.float32)
    zeros = jnp.full((_L,), 0.0, jnp.float32)
    tail = lane < jnp.full((_L,), _IPS - (_NVEC - 1) * _L, jnp.int32)

    for b in range(2):
        for a in range(_T):
            def _zrow(r, _, b=b, a=a):
                for k in range(_C // _L):
                    obuf[b, a, r, pl.ds(k * _L, _L)] = zeros
                return _
            lax.fori_loop(0, _S, _zrow, None)

    def _scatter(step, b, value):
        for v in range(_NVEC):
            k = lane + (v * _L)
            i0 = lax.div(k, s50)
            i1 = lax.rem(k, s50)
            i2 = idxbuf[pl.ds(step * _IPS + v * _L, _L)]
            mask = tail if v == _NVEC - 1 else None
            plsc.store_scatter(obuf.at[b], [i0, i1, i2], value, mask=mask)

    def _out_copy(j, b):
        return pltpu.make_async_copy(
            obuf.at[b], out_hbm.at[pl.ds(base_row + j * _T, _T)], semo.at[b]
        )

    def _do_step(j, b):
        @pl.when(j >= 2)
        def _reclaim():
            _out_copy(j - 2, b).wait()
            _scatter(j - 2, b, zeros)

        _scatter(j, b, ones)
        _out_copy(j, b).start()

    def _step(jj, _):
        _do_step(jj * 2, 0)
        _do_step(jj * 2 + 1, 1)
        return _

    lax.fori_loop(0, _STEPS // 2, _step, None)
    _out_copy(_STEPS - 2, 0).wait()
    _out_copy(_STEPS - 1, 1).wait()


def kernel(x):
    mesh = plsc.VectorSubcoreMesh(
        core_axis_name="c", subcore_axis_name="s",
        num_cores=_NC, num_subcores=_NS,
    )
    sc_onehot = pl.kernel(
        _sc_body,
        out_type=jax.ShapeDtypeStruct((_B, _S, _C), jnp.float32),
        mesh=mesh,
        scratch_types=[
            pltpu.VMEM((2, _T, _S, _C), jnp.float32),
            pltpu.VMEM((_RPW * _S + _L,), jnp.int32),
            pltpu.SemaphoreType.DMA((2,)),
        ],
        compiler_params=pltpu.CompilerParams(
            use_tc_tiling_on_sc=True, needs_layout_passes=False,
        ),
    )
    return sc_onehot(x.reshape(-1).astype(jnp.int32))
